# speedup vs baseline: 4.2721x; 4.2721x over previous
import functools

import jax
import jax.numpy as jnp
from jax import lax
from jax.experimental import pallas as pl
from jax.experimental.pallas import tpu as pltpu
from jax.experimental.pallas import tpu_sc as plsc

N = 100000
D = 128
G = 512

C = 128
FULL_CHUNKS = N // C
TAIL = N - FULL_CHUNKS * C
NW = 32
BASE_CHUNKS = FULL_CHUNKS // NW
EXTRA_WORKERS = FULL_CHUNKS - BASE_CHUNKS * NW

_mesh = plsc.VectorSubcoreMesh(core_axis_name="c", subcore_axis_name="s")


@functools.partial(
    pl.kernel,
    out_type=jax.ShapeDtypeStruct((2, G, D), jnp.float32),
    mesh=_mesh,
    scratch_types=[
        pltpu.VMEM((C,), jnp.int32),
        pltpu.VMEM((C, D), jnp.float32),
        pltpu.VMEM((TAIL,), jnp.int32),
        pltpu.VMEM((TAIL, D), jnp.float32),
        pltpu.VMEM_SHARED((G, D), jnp.float32),
    ],
)
def _sc_segment_sum(x_hbm, b_hbm, zeros_hbm, out_hbm,
                    idx_v, rows_v, tidx_v, trows_v, acc_sh):
    cid = lax.axis_index("c")
    sid = lax.axis_index("s")
    wid = cid * 16 + sid

    @pl.when(sid == 0)
    def _():
        pltpu.sync_copy(zeros_hbm, acc_sh)

    plsc.subcore_barrier()

    c0 = BASE_CHUNKS * wid + jnp.minimum(wid, EXTRA_WORKERS)

    def do_chunk(c):
        base = c * C
        pltpu.sync_copy(b_hbm.at[pl.ds(base, C)], idx_v)
        pltpu.sync_copy(x_hbm.at[pl.ds(base, C)], rows_v)
        pltpu.sync_copy(rows_v, acc_sh.at[idx_v], add=True)

    def body(k, carry):
        do_chunk(c0 + k)
        return carry

    lax.fori_loop(0, BASE_CHUNKS, body, 0)

    @pl.when(wid < EXTRA_WORKERS)
    def _():
        do_chunk(c0 + BASE_CHUNKS)

    @pl.when(wid == NW - 1)
    def _():
        tbase = FULL_CHUNKS * C
        pltpu.sync_copy(b_hbm.at[pl.ds(tbase, TAIL)], tidx_v)
        pltpu.sync_copy(x_hbm.at[pl.ds(tbase, TAIL)], trows_v)
        pltpu.sync_copy(trows_v, acc_sh.at[tidx_v], add=True)

    plsc.subcore_barrier()

    rows_per_tile = G // 16
    pltpu.sync_copy(acc_sh.at[pl.ds(sid * rows_per_tile, rows_per_tile)],
                    out_hbm.at[cid, pl.ds(sid * rows_per_tile, rows_per_tile)])


def _combine_body(p_ref, o_ref):
    o_ref[...] = p_ref[0] + p_ref[1]


_combine = pl.pallas_call(
    _combine_body,
    out_shape=jax.ShapeDtypeStruct((G, D), jnp.float32),
)


def kernel(input, batch, num_graphs):
    b = batch.astype(jnp.int32)
    zeros = jnp.zeros((G, D), jnp.float32)
    partials = _sc_segment_sum(input, b, zeros)
    return _combine(partials)


# speedup vs baseline: 6.5226x; 1.5268x over previous
import functools

import jax
import jax.numpy as jnp
from jax import lax
from jax.experimental import pallas as pl
from jax.experimental.pallas import tpu as pltpu
from jax.experimental.pallas import tpu_sc as plsc

N = 100000
D = 128
G = 512

C = 128
FULL_CHUNKS = N // C
TAIL = N - FULL_CHUNKS * C
NW = 32
CPW = 25
ROWS_PER_TILE = G // 16

_mesh = plsc.VectorSubcoreMesh(core_axis_name="c", subcore_axis_name="s")


@functools.partial(
    pl.kernel,
    out_type=jax.ShapeDtypeStruct((2, G, D), jnp.float32),
    mesh=_mesh,
    scratch_types=[
        pltpu.VMEM((CPW, C), jnp.int32),
        pltpu.VMEM((C, D), jnp.float32),
        pltpu.VMEM((C, D), jnp.float32),
        pltpu.VMEM((TAIL,), jnp.int32),
        pltpu.VMEM((TAIL, D), jnp.float32),
        pltpu.VMEM_SHARED((G, D), jnp.float32),
        pltpu.SemaphoreType.DMA,
        pltpu.SemaphoreType.DMA,
    ],
)
def _sc_segment_sum(x_hbm, b_hbm, b3d_hbm, zeros_hbm, out_hbm,
                    idx2d_v, r0_v, r1_v, tidx_v, trows_v, acc_sh,
                    sem0, sem1):
    cid = lax.axis_index("c")
    sid = lax.axis_index("s")
    wid = cid * 16 + sid
    g0 = wid * CPW

    pltpu.sync_copy(zeros_hbm.at[pl.ds(sid * ROWS_PER_TILE, ROWS_PER_TILE)],
                    acc_sh.at[pl.ds(sid * ROWS_PER_TILE, ROWS_PER_TILE)])
    pltpu.sync_copy(b3d_hbm.at[wid], idx2d_v)
    plsc.subcore_barrier()

    def valid(c):
        return g0 + c < FULL_CHUNKS

    def start(c, buf, sem):
        @pl.when(valid(c))
        def _():
            pltpu.async_copy(x_hbm.at[pl.ds((g0 + c) * C, C)], buf, sem)

    def finish(c, buf, sem):
        @pl.when(valid(c))
        def _():
            pltpu.make_async_copy(x_hbm.at[pl.ds((g0 + c) * C, C)], buf,
                                  sem).wait()
            pltpu.sync_copy(buf, acc_sh.at[idx2d_v.at[c]], add=True)

    start(0, r0_v, sem0)
    start(1, r1_v, sem1)

    def body(i, carry):
        finish(2 * i, r0_v, sem0)
        start(2 * i + 2, r0_v, sem0)

        finish(2 * i + 1, r1_v, sem1)

        @pl.when(i < CPW // 2 - 1)
        def _():
            start(2 * i + 3, r1_v, sem1)

        return carry

    lax.fori_loop(0, CPW // 2, body, 0)
    finish(CPW - 1, r0_v, sem0)

    @pl.when(wid == NW - 1)
    def _():
        tbase = FULL_CHUNKS * C
        pltpu.sync_copy(b_hbm.at[pl.ds(tbase, TAIL)], tidx_v)
        pltpu.sync_copy(x_hbm.at[pl.ds(tbase, TAIL)], trows_v)
        pltpu.sync_copy(trows_v, acc_sh.at[tidx_v], add=True)

    plsc.subcore_barrier()

    pltpu.sync_copy(
        acc_sh.at[pl.ds(sid * ROWS_PER_TILE, ROWS_PER_TILE)],
        out_hbm.at[cid, pl.ds(sid * ROWS_PER_TILE, ROWS_PER_TILE)])


def _combine_body(p_ref, o_ref):
    o_ref[...] = p_ref[0] + p_ref[1]


_combine = pl.pallas_call(
    _combine_body,
    out_shape=jax.ShapeDtypeStruct((G, D), jnp.float32),
)


def kernel(input, batch, num_graphs):
    b = batch.astype(jnp.int32)
    b3d = jnp.pad(b, (0, NW * CPW * C - N)).reshape(NW, CPW, C)
    zeros = jnp.zeros((G, D), jnp.float32)
    partials = _sc_segment_sum(input, b, b3d, zeros)
    return _combine(partials)


# speedup vs baseline: 6.8022x; 1.0429x over previous
import functools

import jax
import jax.numpy as jnp
from jax import lax
from jax.experimental import pallas as pl
from jax.experimental.pallas import tpu as pltpu
from jax.experimental.pallas import tpu_sc as plsc

N = 100000
D = 128
G = 512

C = 128
FULL_CHUNKS = N // C
TAIL = N - FULL_CHUNKS * C
NW = 32
NBUF = 5
ROUNDS = 5
CPW = NBUF * ROUNDS
ROWS_PER_TILE = G // 16

_mesh = plsc.VectorSubcoreMesh(core_axis_name="c", subcore_axis_name="s")

_scratch = (
    [pltpu.VMEM((C, D), jnp.float32) for _ in range(NBUF)]
    + [pltpu.VMEM((C,), jnp.int32) for _ in range(NBUF)]
    + [pltpu.VMEM((TAIL,), jnp.int32),
       pltpu.VMEM((TAIL, D), jnp.float32),
       pltpu.VMEM((ROWS_PER_TILE, D), jnp.float32),
       pltpu.VMEM_SHARED((G, D), jnp.float32)]
    + [pltpu.SemaphoreType.DMA for _ in range(3 * NBUF)]
)


@functools.partial(
    pl.kernel,
    out_type=jax.ShapeDtypeStruct((2, G, D), jnp.float32),
    mesh=_mesh,
    scratch_types=_scratch,
)
def _sc_segment_sum(x_hbm, b_hbm, out_hbm, *refs):
    r_v = refs[0:NBUF]
    i_v = refs[NBUF:2 * NBUF]
    tidx_v, trows_v, z_v, acc_sh = refs[2 * NBUF:2 * NBUF + 4]
    rsem = refs[2 * NBUF + 4:2 * NBUF + 4 + NBUF]
    isem = refs[2 * NBUF + 4 + NBUF:2 * NBUF + 4 + 2 * NBUF]
    ssem = refs[2 * NBUF + 4 + 2 * NBUF:]

    cid = lax.axis_index("c")
    sid = lax.axis_index("s")
    wid = cid * 16 + sid
    g0 = wid * CPW

    def valid(c):
        return g0 + c < FULL_CHUNKS

    def load(c, b):
        @pl.when(valid(c))
        def _():
            base = (g0 + c) * C
            pltpu.async_copy(b_hbm.at[pl.ds(base, C)], i_v[b], isem[b])
            pltpu.async_copy(x_hbm.at[pl.ds(base, C)], r_v[b], rsem[b])

    def process(c, b):
        @pl.when(valid(c))
        def _():
            base = (g0 + c) * C
            pltpu.make_async_copy(b_hbm.at[pl.ds(base, C)], i_v[b],
                                  isem[b]).wait()
            pltpu.make_async_copy(x_hbm.at[pl.ds(base, C)], r_v[b],
                                  rsem[b]).wait()
            pltpu.async_copy(r_v[b], acc_sh.at[i_v[b]], ssem[b], add=True)

    def drain(c, b):
        @pl.when(valid(c))
        def _():
            pltpu.make_async_copy(r_v[b], acc_sh.at[i_v[b]], ssem[b]).wait()

    for b in range(NBUF):
        load(b, b)

    for j in range(ROWS_PER_TILE):
        for k in range(D // 16):
            z_v[j, pl.ds(k * 16, 16)] = jnp.zeros((16,), jnp.float32)
    pltpu.sync_copy(z_v, acc_sh.at[pl.ds(sid * ROWS_PER_TILE, ROWS_PER_TILE)])
    plsc.subcore_barrier()

    def round_body(r, carry):
        for b in range(NBUF):
            process(NBUF * r + b, b)
        for b in range(NBUF):
            @pl.when(r < ROUNDS - 1)
            def _():
                drain(NBUF * r + b, b)
                load(NBUF * (r + 1) + b, b)
        return carry

    lax.fori_loop(0, ROUNDS, round_body, 0)
    for b in range(NBUF):
        drain(NBUF * (ROUNDS - 1) + b, b)

    @pl.when(wid == NW - 1)
    def _():
        tbase = FULL_CHUNKS * C
        pltpu.sync_copy(b_hbm.at[pl.ds(tbase, TAIL)], tidx_v)
        pltpu.sync_copy(x_hbm.at[pl.ds(tbase, TAIL)], trows_v)
        pltpu.sync_copy(trows_v, acc_sh.at[tidx_v], add=True)

    plsc.subcore_barrier()

    pltpu.sync_copy(
        acc_sh.at[pl.ds(sid * ROWS_PER_TILE, ROWS_PER_TILE)],
        out_hbm.at[cid, pl.ds(sid * ROWS_PER_TILE, ROWS_PER_TILE)])


def _combine_body(p_ref, o_ref):
    o_ref[...] = p_ref[0] + p_ref[1]


_combine = pl.pallas_call(
    _combine_body,
    out_shape=jax.ShapeDtypeStruct((G, D), jnp.float32),
)


def kernel(input, batch, num_graphs):
    partials = _sc_segment_sum(input, batch.astype(jnp.int32))
    return _combine(partials)
